# baseline (device time: 77390 ns/iter reference)
import jax
import jax.numpy as jnp
from jax import lax
from jax.experimental import pallas as pl
from jax.experimental.pallas import tpu as pltpu

N_DEV = 16


def kernel(A, B):
    m, k = A.shape
    _, n = B.shape
    m_per = m // N_DEV

    def body(a_ref, b_ref, out_ref, comm_ref, send_sems, recv_sems):
        my = lax.axis_index("i")
        left = (my + N_DEV - 1) % N_DEV
        right = (my + 1) % N_DEV

        barrier_sem = pltpu.get_barrier_semaphore()
        for nbr in (left, right):
            pl.semaphore_signal(
                barrier_sem, inc=1,
                device_id=(nbr,), device_id_type=pl.DeviceIdType.MESH,
            )
        pl.semaphore_wait(barrier_sem, 2)

        def partial_chunk(c):
            return jnp.dot(
                a_ref[pl.ds(c * m_per, m_per), :], b_ref[:, :],
                preferred_element_type=jnp.float32,
            )

        comm_ref[0, :, :] = partial_chunk((my + N_DEV - 1) % N_DEV)

        for h in range(N_DEV - 1):
            rdma = pltpu.make_async_remote_copy(
                src_ref=comm_ref.at[h],
                dst_ref=comm_ref.at[h + 1],
                send_sem=send_sems.at[h],
                recv_sem=recv_sems.at[h],
                device_id=(right,),
                device_id_type=pl.DeviceIdType.MESH,
            )
            rdma.start()
            part = partial_chunk((my + 2 * N_DEV - h - 2) % N_DEV)
            rdma.wait()
            comm_ref[h + 1, :, :] = comm_ref[h + 1, :, :] + part

        out_ref[:, :] = comm_ref[N_DEV - 1, :, :]

    return pl.pallas_call(
        body,
        out_shape=jax.ShapeDtypeStruct((m_per, n), jnp.float32),
        in_specs=[
            pl.BlockSpec(memory_space=pltpu.VMEM),
            pl.BlockSpec(memory_space=pltpu.VMEM),
        ],
        out_specs=pl.BlockSpec(memory_space=pltpu.VMEM),
        scratch_shapes=[
            pltpu.VMEM((N_DEV, m_per, n), jnp.float32),
            pltpu.SemaphoreType.DMA((N_DEV - 1,)),
            pltpu.SemaphoreType.DMA((N_DEV - 1,)),
        ],
        compiler_params=pltpu.CompilerParams(collective_id=0),
    )(A, B)


# device time: 41947 ns/iter; 1.8449x vs baseline; 1.8449x over previous
import jax
import jax.numpy as jnp
from jax import lax
from jax.experimental import pallas as pl
from jax.experimental.pallas import tpu as pltpu

N_DEV = 16
NP = 4
NZ = 4
MP = 64


def kernel(A, B):
    m, k = A.shape
    _, n = B.shape
    nh = n // 2

    def body(a_ref, b_ref, out_ref,
             ring_p, ring_m, recv_p, recv_m,
             ps_sems, pr_sems, ms_sems, mr_sems,
             zs_p, zr_p, zs_m, zr_m):
        i = lax.axis_index("i")
        z = i // NP
        p = i % NP
        zbase = z * NP
        plane_r = zbase + (p + 1) % NP
        plane_l = zbase + (p + NP - 1) % NP

        barrier = pltpu.get_barrier_semaphore()
        peers = [plane_r, plane_l] + [((z + dz) % NZ) * NP + p for dz in (1, 2, 3)]
        for tgt in peers:
            pl.semaphore_signal(
                barrier, inc=1,
                device_id=(tgt,), device_id_type=pl.DeviceIdType.MESH,
            )
        pl.semaphore_wait(barrier, 5)

        def partial(q, zp, col0):
            return jnp.dot(
                a_ref[pl.ds(zp * (NP * MP) + q * MP, MP), :],
                b_ref[:, col0:col0 + nh],
                preferred_element_type=jnp.float32,
            )

        qp0 = (p + NP - 1) % NP
        qm0 = (p + 1) % NP
        for zp in range(NZ):
            ring_p[0, zp, :, :] = partial(qp0, zp, 0)
            ring_m[0, zp, :, :] = partial(qm0, zp, nh)

        for h in range(NP - 1):
            rp = pltpu.make_async_remote_copy(
                src_ref=ring_p.at[h], dst_ref=ring_p.at[h + 1],
                send_sem=ps_sems.at[h], recv_sem=pr_sems.at[h],
                device_id=(plane_r,), device_id_type=pl.DeviceIdType.MESH,
            )
            rm = pltpu.make_async_remote_copy(
                src_ref=ring_m.at[h], dst_ref=ring_m.at[h + 1],
                send_sem=ms_sems.at[h], recv_sem=mr_sems.at[h],
                device_id=(plane_l,), device_id_type=pl.DeviceIdType.MESH,
            )
            rp.start()
            rm.start()
            qpr = (p + 2 * NP - h - 2) % NP
            qmr = (p + h + 2) % NP
            parts_p = [partial(qpr, zp, 0) for zp in range(NZ)]
            parts_m = [partial(qmr, zp, nh) for zp in range(NZ)]
            rp.wait()
            rm.wait()
            for zp in range(NZ):
                ring_p[h + 1, zp, :, :] = ring_p[h + 1, zp, :, :] + parts_p[zp]
                ring_m[h + 1, zp, :, :] = ring_m[h + 1, zp, :, :] + parts_m[zp]

        last = NP - 1

        sends = []
        for dz in (1, 2, 3):
            zt = (z + dz) % NZ
            tgt = zt * NP + p
            s1 = pltpu.make_async_remote_copy(
                src_ref=ring_p.at[last, zt], dst_ref=recv_p.at[z],
                send_sem=zs_p.at[dz - 1], recv_sem=zr_p.at[z],
                device_id=(tgt,), device_id_type=pl.DeviceIdType.MESH,
            )
            s2 = pltpu.make_async_remote_copy(
                src_ref=ring_m.at[last, zt], dst_ref=recv_m.at[z],
                send_sem=zs_m.at[dz - 1], recv_sem=zr_m.at[z],
                device_id=(tgt,), device_id_type=pl.DeviceIdType.MESH,
            )
            s1.start()
            s2.start()
            sends += [s1, s2]

        acc_p = ring_p[last, z, :, :]
        acc_m = ring_m[last, z, :, :]
        for dz in (1, 2, 3):
            zs = (z + dz) % NZ
            r1 = pltpu.make_async_remote_copy(
                src_ref=ring_p.at[last, 0], dst_ref=recv_p.at[zs],
                send_sem=zs_p.at[dz - 1], recv_sem=zr_p.at[zs],
                device_id=(i,), device_id_type=pl.DeviceIdType.MESH,
            )
            r2 = pltpu.make_async_remote_copy(
                src_ref=ring_m.at[last, 0], dst_ref=recv_m.at[zs],
                send_sem=zs_m.at[dz - 1], recv_sem=zr_m.at[zs],
                device_id=(i,), device_id_type=pl.DeviceIdType.MESH,
            )
            r1.wait_recv()
            r2.wait_recv()
            acc_p = acc_p + recv_p[zs, :, :]
            acc_m = acc_m + recv_m[zs, :, :]

        out_ref[:, 0:nh] = acc_p
        out_ref[:, nh:n] = acc_m

        for s in sends:
            s.wait_send()

    return pl.pallas_call(
        body,
        out_shape=jax.ShapeDtypeStruct((MP, n), jnp.float32),
        in_specs=[
            pl.BlockSpec(memory_space=pltpu.VMEM),
            pl.BlockSpec(memory_space=pltpu.VMEM),
        ],
        out_specs=pl.BlockSpec(memory_space=pltpu.VMEM),
        scratch_shapes=[
            pltpu.VMEM((NP, NZ, MP, nh), jnp.float32),
            pltpu.VMEM((NP, NZ, MP, nh), jnp.float32),
            pltpu.VMEM((NZ, MP, nh), jnp.float32),
            pltpu.VMEM((NZ, MP, nh), jnp.float32),
            pltpu.SemaphoreType.DMA((NP - 1,)),
            pltpu.SemaphoreType.DMA((NP - 1,)),
            pltpu.SemaphoreType.DMA((NP - 1,)),
            pltpu.SemaphoreType.DMA((NP - 1,)),
            pltpu.SemaphoreType.DMA((NZ - 1,)),
            pltpu.SemaphoreType.DMA((NZ,)),
            pltpu.SemaphoreType.DMA((NZ - 1,)),
            pltpu.SemaphoreType.DMA((NZ,)),
        ],
        compiler_params=pltpu.CompilerParams(collective_id=0),
    )(A, B)


# device time: 36846 ns/iter; 2.1004x vs baseline; 1.1384x over previous
import jax
import jax.numpy as jnp
from jax import lax
from jax.experimental import pallas as pl
from jax.experimental.pallas import tpu as pltpu

N_DEV = 16
NP = 4
NZ = 4
MP = 64
NQ = 4


def kernel(A, B):
    m, k = A.shape
    _, n = B.shape
    qw = n // NQ

    def body(a_ref, b_ref, out_ref, ring, zrecv,
             rs_sems, rr_sems, zs_sems, zr_sems):
        i = lax.axis_index("i")
        z = i // NP
        p = i % NP
        zbase = z * NP
        plane_r = zbase + (p + 1) % NP
        plane_l = zbase + (p + NP - 1) % NP

        barrier = pltpu.get_barrier_semaphore()
        peers = [plane_r, plane_l] + [((z + dz) % NZ) * NP + p for dz in (1, 2, 3)]
        for tgt in peers:
            pl.semaphore_signal(
                barrier, inc=1,
                device_id=(tgt,), device_id_type=pl.DeviceIdType.MESH,
            )

        def partial(qset, zp, q):
            return jnp.dot(
                a_ref[pl.ds(zp * (NP * MP) + qset * MP, MP), :],
                b_ref[:, q * qw:(q + 1) * qw],
                preferred_element_type=jnp.float32,
            )

        def qsend0(q):
            return (p + NP - 1) % NP if q < 2 else (p + 1) % NP

        def qrecv(q, h):
            return (p + 2 * NP - h - 2) % NP if q < 2 else (p + h + 2) % NP

        for q in range(NQ):
            qs = qsend0(q)
            for zp in range(NZ):
                ring[q, 0, zp, :, :] = partial(qs, zp, q)

        pl.semaphore_wait(barrier, 5)

        def mk_ring_rdma(q, h):
            tgt = plane_r if q < 2 else plane_l
            return pltpu.make_async_remote_copy(
                src_ref=ring.at[q, h], dst_ref=ring.at[q, h + 1],
                send_sem=rs_sems.at[q, h], recv_sem=rr_sems.at[q, h],
                device_id=(tgt,), device_id_type=pl.DeviceIdType.MESH,
            )

        order = (0, 2, 1, 3)
        hop_rdmas = {}
        for q in order:
            r = mk_ring_rdma(q, 0)
            r.start()
            hop_rdmas[q] = r

        phase2_sends = []
        for h in range(NP - 1):
            for q in order:
                qr = qrecv(q, h)
                parts = [partial(qr, zp, q) for zp in range(NZ)]
                hop_rdmas[q].wait()
                for zp in range(NZ):
                    ring[q, h + 1, zp, :, :] = ring[q, h + 1, zp, :, :] + parts[zp]
                if h < NP - 2:
                    r = mk_ring_rdma(q, h + 1)
                    r.start()
                    hop_rdmas[q] = r
                else:
                    for dz in (1, 2, 3):
                        zt = (z + dz) % NZ
                        s = pltpu.make_async_remote_copy(
                            src_ref=ring.at[q, NP - 1, zt],
                            dst_ref=zrecv.at[q, z],
                            send_sem=zs_sems.at[q, dz - 1],
                            recv_sem=zr_sems.at[q, z],
                            device_id=(zt * NP + p,),
                            device_id_type=pl.DeviceIdType.MESH,
                        )
                        s.start()
                        phase2_sends.append(s)

        for q in order:
            acc = ring[q, NP - 1, z, :, :]
            for dz in (1, 2, 3):
                zs = (z + dz) % NZ
                rwait = pltpu.make_async_remote_copy(
                    src_ref=ring.at[q, NP - 1, 0], dst_ref=zrecv.at[q, zs],
                    send_sem=zs_sems.at[q, dz - 1], recv_sem=zr_sems.at[q, zs],
                    device_id=(i,), device_id_type=pl.DeviceIdType.MESH,
                )
                rwait.wait_recv()
                acc = acc + zrecv[q, zs, :, :]
            out_ref[:, q * qw:(q + 1) * qw] = acc

        for s in phase2_sends:
            s.wait_send()

    return pl.pallas_call(
        body,
        out_shape=jax.ShapeDtypeStruct((MP, n), jnp.float32),
        in_specs=[
            pl.BlockSpec(memory_space=pltpu.VMEM),
            pl.BlockSpec(memory_space=pltpu.VMEM),
        ],
        out_specs=pl.BlockSpec(memory_space=pltpu.VMEM),
        scratch_shapes=[
            pltpu.VMEM((NQ, NP, NZ, MP, qw), jnp.float32),
            pltpu.VMEM((NQ, NZ, MP, qw), jnp.float32),
            pltpu.SemaphoreType.DMA((NQ, NP - 1)),
            pltpu.SemaphoreType.DMA((NQ, NP - 1)),
            pltpu.SemaphoreType.DMA((NQ, NZ - 1)),
            pltpu.SemaphoreType.DMA((NQ, NZ)),
        ],
        compiler_params=pltpu.CompilerParams(collective_id=0),
    )(A, B)


# device time: 36820 ns/iter; 2.1018x vs baseline; 1.0007x over previous
import jax
import jax.numpy as jnp
from jax import lax
from jax.experimental import pallas as pl
from jax.experimental.pallas import tpu as pltpu

N_DEV = 16
NP = 4
NZ = 4
MP = 64
NQ = 4


def kernel(A, B):
    m, k = A.shape
    _, n = B.shape
    qw = n // NQ

    def body(a_ref, b_ref, out_ref,
             sd, sr, sc, rd, rr, rc, ps, zrecv,
             sd_s, sd_r, sr_s, sr_r, sc_s, sc_r, zs_sems, zr_sems):
        i = lax.axis_index("i")
        z = i // NP
        p = i % NP
        zbase = z * NP
        plane_r = zbase + (p + 1) % NP
        plane_l = zbase + (p + NP - 1) % NP

        barrier = pltpu.get_barrier_semaphore()
        peers = [plane_r, plane_l] + [((z + dz) % NZ) * NP + p for dz in (1, 2, 3)]
        for tgt in peers:
            pl.semaphore_signal(
                barrier, inc=1,
                device_id=(tgt,), device_id_type=pl.DeviceIdType.MESH,
            )

        def partial(qset, zp, q):
            return jnp.dot(
                a_ref[pl.ds(zp * (NP * MP) + qset * MP, MP), :],
                b_ref[:, q * qw:(q + 1) * qw],
                preferred_element_type=jnp.float32,
            )

        def cw(q):
            return q % 2 == 0

        def qd(q):
            return (p + 1) % NP if cw(q) else (p + NP - 1) % NP

        def qr(q):
            return (p + 2) % NP

        def qc(q):
            return (p + NP - 1) % NP if cw(q) else (p + 1) % NP

        def mk(src, dst, ssem, rsem, q, tgt):
            return pltpu.make_async_remote_copy(
                src_ref=src.at[q], dst_ref=dst.at[q],
                send_sem=ssem.at[q], recv_sem=rsem.at[q],
                device_id=(tgt,), device_id_type=pl.DeviceIdType.MESH,
            )

        for q in range(NQ):
            for zp in range(NZ):
                sr[q, zp, :, :] = partial(qr(q), zp, q)

        pl.semaphore_wait(barrier, 5)

        relay = [mk(sr, rr, sr_s, sr_r, q, plane_l if cw(q) else plane_r)
                 for q in range(NQ)]
        direct = [mk(sd, rd, sd_s, sd_r, q, plane_r if cw(q) else plane_l)
                  for q in range(NQ)]
        comb = [mk(sc, rc, sc_s, sc_r, q, plane_l if cw(q) else plane_r)
                for q in range(NQ)]

        for q in range(NQ):
            relay[q].start()

        for q in range(NQ):
            for zp in range(NZ):
                sd[q, zp, :, :] = partial(qd(q), zp, q)
            direct[q].start()

        for q in range(NQ):
            for zp in range(NZ):
                sc[q, zp, :, :] = partial(qc(q), zp, q)

        for q in range(NQ):
            relay[q].wait_recv()
            for zp in range(NZ):
                sc[q, zp, :, :] = sc[q, zp, :, :] + rr[q, zp, :, :]
            comb[q].start()

        for q in range(NQ):
            for zp in range(NZ):
                ps[q, zp, :, :] = partial(p, zp, q)

        zsends = []
        for q in range(NQ):
            direct[q].wait_recv()
            comb[q].wait_recv()
            for zp in range(NZ):
                ps[q, zp, :, :] = ps[q, zp, :, :] + rd[q, zp, :, :] + rc[q, zp, :, :]
            for dz in (1, 2, 3):
                zt = (z + dz) % NZ
                s = pltpu.make_async_remote_copy(
                    src_ref=ps.at[q, zt], dst_ref=zrecv.at[q, z],
                    send_sem=zs_sems.at[q, dz - 1], recv_sem=zr_sems.at[q, z],
                    device_id=(zt * NP + p,),
                    device_id_type=pl.DeviceIdType.MESH,
                )
                s.start()
                zsends.append(s)

        for q in range(NQ):
            acc = ps[q, z, :, :]
            for dz in (1, 2, 3):
                zs = (z + dz) % NZ
                rwait = pltpu.make_async_remote_copy(
                    src_ref=ps.at[q, 0], dst_ref=zrecv.at[q, zs],
                    send_sem=zs_sems.at[q, dz - 1], recv_sem=zr_sems.at[q, zs],
                    device_id=(i,), device_id_type=pl.DeviceIdType.MESH,
                )
                rwait.wait_recv()
                acc = acc + zrecv[q, zs, :, :]
            out_ref[:, q * qw:(q + 1) * qw] = acc

        for q in range(NQ):
            relay[q].wait_send()
            direct[q].wait_send()
            comb[q].wait_send()
        for s in zsends:
            s.wait_send()

    buf = pltpu.VMEM((NQ, NZ, MP, qw), jnp.float32)
    return pl.pallas_call(
        body,
        out_shape=jax.ShapeDtypeStruct((MP, n), jnp.float32),
        in_specs=[
            pl.BlockSpec(memory_space=pltpu.VMEM),
            pl.BlockSpec(memory_space=pltpu.VMEM),
        ],
        out_specs=pl.BlockSpec(memory_space=pltpu.VMEM),
        scratch_shapes=[
            buf, buf, buf, buf, buf, buf, buf,
            pltpu.VMEM((NQ, NZ, MP, qw), jnp.float32),
            pltpu.SemaphoreType.DMA((NQ,)),
            pltpu.SemaphoreType.DMA((NQ,)),
            pltpu.SemaphoreType.DMA((NQ,)),
            pltpu.SemaphoreType.DMA((NQ,)),
            pltpu.SemaphoreType.DMA((NQ,)),
            pltpu.SemaphoreType.DMA((NQ,)),
            pltpu.SemaphoreType.DMA((NQ, NZ - 1)),
            pltpu.SemaphoreType.DMA((NQ, NZ)),
        ],
        compiler_params=pltpu.CompilerParams(collective_id=0),
    )(A, B)


# device time: 34109 ns/iter; 2.2689x vs baseline; 1.0795x over previous
import jax
import jax.numpy as jnp
from jax import lax
from jax.experimental import pallas as pl
from jax.experimental.pallas import tpu as pltpu

N_DEV = 16
NP = 4
NZ = 4
MP = 64
NQ = 4


def kernel(A, B):
    m, k = A.shape
    _, n = B.shape
    qw = n // NQ

    def body(a_ref, b_ref, out_ref,
             sd, sr, sc, rd, rr, rc, ps, zrecv,
             sd_s, sd_r, sr_s, sr_r, sc_s, sc_r, zs_sems, zr_sems):
        i = lax.axis_index("i")
        z = i // NP
        p = i % NP
        zbase = z * NP
        plane_r = zbase + (p + 1) % NP
        plane_l = zbase + (p + NP - 1) % NP

        barrier = pltpu.get_barrier_semaphore()
        peers = [plane_r, plane_l] + [((z + dz) % NZ) * NP + p for dz in (1, 2, 3)]
        for tgt in peers:
            pl.semaphore_signal(
                barrier, inc=1,
                device_id=(tgt,), device_id_type=pl.DeviceIdType.MESH,
            )

        def partial(qset, zp, q):
            return jnp.dot(
                a_ref[pl.ds(zp * (NP * MP) + qset * MP, MP), :],
                b_ref[:, q * qw:(q + 1) * qw],
                preferred_element_type=jnp.float32,
            )

        def cw(q):
            return q % 2 == 0

        def qd(q):
            return (p + 1) % NP if cw(q) else (p + NP - 1) % NP

        def qr(q):
            return (p + 2) % NP

        def qc(q):
            return (p + NP - 1) % NP if cw(q) else (p + 1) % NP

        def mk(src, dst, ssem, rsem, q, tgt):
            return pltpu.make_async_remote_copy(
                src_ref=src.at[q], dst_ref=dst.at[q],
                send_sem=ssem.at[q], recv_sem=rsem.at[q],
                device_id=(tgt,), device_id_type=pl.DeviceIdType.MESH,
            )

        for q in range(NQ):
            for zp in range(NZ):
                sr[q, zp, :, :] = partial(qr(q), zp, q)

        pl.semaphore_wait(barrier, 5)

        relay = [mk(sr, rr, sr_s, sr_r, q, plane_l if cw(q) else plane_r)
                 for q in range(NQ)]
        direct = [mk(sd, rd, sd_s, sd_r, q, plane_r if cw(q) else plane_l)
                  for q in range(NQ)]
        comb = [mk(sc, rc, sc_s, sc_r, q, plane_l if cw(q) else plane_r)
                for q in range(NQ)]

        for q in range(NQ):
            relay[q].start()

        for pair in ((0, 1), (2, 3)):
            for q in pair:
                for zp in range(NZ):
                    sc[q, zp, :, :] = partial(qc(q), zp, q)
            for q in pair:
                relay[q].wait_recv()
                for zp in range(NZ):
                    sc[q, zp, :, :] = sc[q, zp, :, :] + rr[q, zp, :, :]
                comb[q].start()
            for q in pair:
                for zp in range(NZ):
                    sd[q, zp, :, :] = partial(qd(q), zp, q)
                direct[q].start()

        for q in range(NQ):
            for zp in range(NZ):
                ps[q, zp, :, :] = partial(p, zp, q)

        zsends = []
        for q in range(NQ):
            direct[q].wait_recv()
            comb[q].wait_recv()
            for zp in range(NZ):
                ps[q, zp, :, :] = ps[q, zp, :, :] + rd[q, zp, :, :] + rc[q, zp, :, :]
            for dz in (1, 2, 3):
                zt = (z + dz) % NZ
                s = pltpu.make_async_remote_copy(
                    src_ref=ps.at[q, zt], dst_ref=zrecv.at[q, z],
                    send_sem=zs_sems.at[q, dz - 1], recv_sem=zr_sems.at[q, z],
                    device_id=(zt * NP + p,),
                    device_id_type=pl.DeviceIdType.MESH,
                )
                s.start()
                zsends.append(s)

        for q in range(NQ):
            acc = ps[q, z, :, :]
            for dz in (1, 2, 3):
                zs = (z + dz) % NZ
                rwait = pltpu.make_async_remote_copy(
                    src_ref=ps.at[q, 0], dst_ref=zrecv.at[q, zs],
                    send_sem=zs_sems.at[q, dz - 1], recv_sem=zr_sems.at[q, zs],
                    device_id=(i,), device_id_type=pl.DeviceIdType.MESH,
                )
                rwait.wait_recv()
                acc = acc + zrecv[q, zs, :, :]
            out_ref[:, q * qw:(q + 1) * qw] = acc

        for q in range(NQ):
            relay[q].wait_send()
            direct[q].wait_send()
            comb[q].wait_send()
        for s in zsends:
            s.wait_send()

    buf = pltpu.VMEM((NQ, NZ, MP, qw), jnp.float32)
    return pl.pallas_call(
        body,
        out_shape=jax.ShapeDtypeStruct((MP, n), jnp.float32),
        in_specs=[
            pl.BlockSpec(memory_space=pltpu.VMEM),
            pl.BlockSpec(memory_space=pltpu.VMEM),
        ],
        out_specs=pl.BlockSpec(memory_space=pltpu.VMEM),
        scratch_shapes=[
            buf, buf, buf, buf, buf, buf, buf,
            pltpu.VMEM((NQ, NZ, MP, qw), jnp.float32),
            pltpu.SemaphoreType.DMA((NQ,)),
            pltpu.SemaphoreType.DMA((NQ,)),
            pltpu.SemaphoreType.DMA((NQ,)),
            pltpu.SemaphoreType.DMA((NQ,)),
            pltpu.SemaphoreType.DMA((NQ,)),
            pltpu.SemaphoreType.DMA((NQ,)),
            pltpu.SemaphoreType.DMA((NQ, NZ - 1)),
            pltpu.SemaphoreType.DMA((NQ, NZ)),
        ],
        compiler_params=pltpu.CompilerParams(collective_id=0),
    )(A, B)


# device time: 32004 ns/iter; 2.4181x vs baseline; 1.0658x over previous
import jax
import jax.numpy as jnp
from jax import lax
from jax.experimental import pallas as pl
from jax.experimental.pallas import tpu as pltpu

N_DEV = 16
NP = 4
NZ = 4
MP = 64
NQ = 8


def kernel(A, B):
    m, k = A.shape
    _, n = B.shape
    qw = n // NQ

    def body(a_ref, b_ref, out_ref,
             sd, sr, sc, rd, rr, rc, ps, zrecv,
             sd_s, sd_r, sr_s, sr_r, sc_s, sc_r, zs_sems, zr_sems):
        i = lax.axis_index("i")
        z = i // NP
        p = i % NP
        zbase = z * NP
        plane_r = zbase + (p + 1) % NP
        plane_l = zbase + (p + NP - 1) % NP

        barrier = pltpu.get_barrier_semaphore()
        peers = [plane_r, plane_l] + [((z + dz) % NZ) * NP + p for dz in (1, 2, 3)]
        for tgt in peers:
            pl.semaphore_signal(
                barrier, inc=1,
                device_id=(tgt,), device_id_type=pl.DeviceIdType.MESH,
            )

        def partial(qset, zp, q):
            return jnp.dot(
                a_ref[pl.ds(zp * (NP * MP) + qset * MP, MP), :],
                b_ref[:, q * qw:(q + 1) * qw],
                preferred_element_type=jnp.float32,
            )

        def cw(q):
            return q % 2 == 0

        def qd(q):
            return (p + 1) % NP if cw(q) else (p + NP - 1) % NP

        def qr(q):
            return (p + 2) % NP

        def qc(q):
            return (p + NP - 1) % NP if cw(q) else (p + 1) % NP

        def mk(src, dst, ssem, rsem, q, tgt):
            return pltpu.make_async_remote_copy(
                src_ref=src.at[q], dst_ref=dst.at[q],
                send_sem=ssem.at[q], recv_sem=rsem.at[q],
                device_id=(tgt,), device_id_type=pl.DeviceIdType.MESH,
            )

        for q in range(NQ):
            for zp in range(NZ):
                sr[q, zp, :, :] = partial(qr(q), zp, q)

        pl.semaphore_wait(barrier, 5)

        relay = [mk(sr, rr, sr_s, sr_r, q, plane_l if cw(q) else plane_r)
                 for q in range(NQ)]
        direct = [mk(sd, rd, sd_s, sd_r, q, plane_r if cw(q) else plane_l)
                  for q in range(NQ)]
        comb = [mk(sc, rc, sc_s, sc_r, q, plane_l if cw(q) else plane_r)
                for q in range(NQ)]

        for q in range(NQ):
            relay[q].start()

        for pair in tuple((2 * j, 2 * j + 1) for j in range(NQ // 2)):
            for q in pair:
                for zp in range(NZ):
                    sc[q, zp, :, :] = partial(qc(q), zp, q)
            for q in pair:
                relay[q].wait_recv()
                for zp in range(NZ):
                    sc[q, zp, :, :] = sc[q, zp, :, :] + rr[q, zp, :, :]
                comb[q].start()
            for q in pair:
                for zp in range(NZ):
                    sd[q, zp, :, :] = partial(qd(q), zp, q)
                direct[q].start()

        for q in range(NQ):
            for zp in range(NZ):
                ps[q, zp, :, :] = partial(p, zp, q)

        zsends = []
        for q in range(NQ):
            direct[q].wait_recv()
            comb[q].wait_recv()
            for zp in range(NZ):
                ps[q, zp, :, :] = ps[q, zp, :, :] + rd[q, zp, :, :] + rc[q, zp, :, :]
            for dz in (1, 2, 3):
                zt = (z + dz) % NZ
                s = pltpu.make_async_remote_copy(
                    src_ref=ps.at[q, zt], dst_ref=zrecv.at[q, z],
                    send_sem=zs_sems.at[q, dz - 1], recv_sem=zr_sems.at[q, z],
                    device_id=(zt * NP + p,),
                    device_id_type=pl.DeviceIdType.MESH,
                )
                s.start()
                zsends.append(s)

        for q in range(NQ):
            acc = ps[q, z, :, :]
            for dz in (1, 2, 3):
                zs = (z + dz) % NZ
                rwait = pltpu.make_async_remote_copy(
                    src_ref=ps.at[q, 0], dst_ref=zrecv.at[q, zs],
                    send_sem=zs_sems.at[q, dz - 1], recv_sem=zr_sems.at[q, zs],
                    device_id=(i,), device_id_type=pl.DeviceIdType.MESH,
                )
                rwait.wait_recv()
                acc = acc + zrecv[q, zs, :, :]
            out_ref[:, q * qw:(q + 1) * qw] = acc

        for q in range(NQ):
            relay[q].wait_send()
            direct[q].wait_send()
            comb[q].wait_send()
        for s in zsends:
            s.wait_send()

    buf = pltpu.VMEM((NQ, NZ, MP, qw), jnp.float32)
    return pl.pallas_call(
        body,
        out_shape=jax.ShapeDtypeStruct((MP, n), jnp.float32),
        in_specs=[
            pl.BlockSpec(memory_space=pltpu.VMEM),
            pl.BlockSpec(memory_space=pltpu.VMEM),
        ],
        out_specs=pl.BlockSpec(memory_space=pltpu.VMEM),
        scratch_shapes=[
            buf, buf, buf, buf, buf, buf, buf,
            pltpu.VMEM((NQ, NZ, MP, qw), jnp.float32),
            pltpu.SemaphoreType.DMA((NQ,)),
            pltpu.SemaphoreType.DMA((NQ,)),
            pltpu.SemaphoreType.DMA((NQ,)),
            pltpu.SemaphoreType.DMA((NQ,)),
            pltpu.SemaphoreType.DMA((NQ,)),
            pltpu.SemaphoreType.DMA((NQ,)),
            pltpu.SemaphoreType.DMA((NQ, NZ - 1)),
            pltpu.SemaphoreType.DMA((NQ, NZ)),
        ],
        compiler_params=pltpu.CompilerParams(collective_id=0),
    )(A, B)
